# baseline (device time: 20880 ns/iter reference)
import jax
import jax.numpy as jnp
from jax import lax
from jax.experimental import pallas as pl
from jax.experimental.pallas import tpu as pltpu

N_DEV = 4


def kernel(x, router_W, route_idx, expert_W):
    del router_W
    n, d = x.shape
    e_per, _, h = expert_W.shape

    def body(x_ref, idx_ref, w_ref, out_ref, stage_ref, comm_ref,
             send_sems, recv_sems):
        my_pos = lax.axis_index("i")

        barrier_sem = pltpu.get_barrier_semaphore()
        for o in range(1, N_DEV):
            pl.semaphore_signal(
                barrier_sem, inc=1,
                device_id=((my_pos + o) % N_DEV,),
                device_id_type=pl.DeviceIdType.MESH,
            )
        pl.semaphore_wait(barrier_sem, N_DEV - 1)

        route = idx_ref[:, :]
        xv = x_ref[:, :]
        acc = jnp.zeros((n, h), dtype=jnp.float32)
        for e in range(e_per):
            gid = my_pos * e_per + e
            xm = jnp.where(route == gid, xv, 0.0).astype(jnp.bfloat16)
            acc = acc + jnp.dot(
                xm, w_ref[e, :, :].astype(jnp.bfloat16),
                preferred_element_type=jnp.float32,
            )
        stage_ref[:, :] = acc.astype(jnp.bfloat16)

        sends = []
        for o in range(1, N_DEV):
            rdma = pltpu.make_async_remote_copy(
                src_ref=stage_ref,
                dst_ref=comm_ref.at[o - 1],
                send_sem=send_sems.at[o - 1],
                recv_sem=recv_sems.at[o - 1],
                device_id=((my_pos + o) % N_DEV,),
                device_id_type=pl.DeviceIdType.MESH,
            )
            rdma.start()
            sends.append(rdma)

        out_ref[:, :] = acc

        for o in range(1, N_DEV):
            recv = pltpu.make_async_remote_copy(
                src_ref=stage_ref,
                dst_ref=comm_ref.at[o - 1],
                send_sem=send_sems.at[o - 1],
                recv_sem=recv_sems.at[o - 1],
                device_id=((my_pos + o) % N_DEV,),
                device_id_type=pl.DeviceIdType.MESH,
            )
            recv.wait_recv()
            out_ref[:, :] += comm_ref[o - 1, :, :].astype(jnp.float32)

        for rdma in sends:
            rdma.wait_send()

    return pl.pallas_call(
        body,
        out_shape=jax.ShapeDtypeStruct((n, h), jnp.float32),
        in_specs=[
            pl.BlockSpec(memory_space=pltpu.VMEM),
            pl.BlockSpec(memory_space=pltpu.VMEM),
            pl.BlockSpec(memory_space=pltpu.VMEM),
        ],
        out_specs=pl.BlockSpec(memory_space=pltpu.VMEM),
        scratch_shapes=[
            pltpu.VMEM((n, h), jnp.bfloat16),
            pltpu.VMEM((N_DEV - 1, n, h), jnp.bfloat16),
            pltpu.SemaphoreType.DMA((N_DEV - 1,)),
            pltpu.SemaphoreType.DMA((N_DEV - 1,)),
        ],
        compiler_params=pltpu.CompilerParams(collective_id=0),
    )(x, route_idx, expert_W)


# device time: 17465 ns/iter; 1.1955x vs baseline; 1.1955x over previous
import jax
import jax.numpy as jnp
from jax import lax
from jax.experimental import pallas as pl
from jax.experimental.pallas import tpu as pltpu

N_DEV = 4


def kernel(x, router_W, route_idx, expert_W):
    del router_W
    n, d = x.shape
    e_per, _, h = expert_W.shape
    hc = h // N_DEV

    def body(x_ref, idx_ref, w_ref, out_ref, stage_ref, rs_comm_ref,
             ag_stage_ref, ag_comm_ref, rs_send_sems, rs_recv_sems,
             ag_send_sems, ag_recv_sems):
        my_pos = lax.axis_index("i")

        barrier_sem = pltpu.get_barrier_semaphore()
        for o in range(1, N_DEV):
            pl.semaphore_signal(
                barrier_sem, inc=1,
                device_id=((my_pos + o) % N_DEV,),
                device_id_type=pl.DeviceIdType.MESH,
            )
        pl.semaphore_wait(barrier_sem, N_DEV - 1)

        route = idx_ref[:, :]
        xv = x_ref[:, :]
        acc = jnp.zeros((n, h), dtype=jnp.float32)
        for e in range(e_per):
            gid = my_pos * e_per + e
            xm = jnp.where(route == gid, xv, 0.0).astype(jnp.bfloat16)
            acc = acc + jnp.dot(
                xm, w_ref[e, :, :].astype(jnp.bfloat16),
                preferred_element_type=jnp.float32,
            )
        accb = acc.astype(jnp.bfloat16)
        for c in range(N_DEV):
            stage_ref[c, :, :] = accb[:, c * hc:(c + 1) * hc]

        rs_sends = []
        for o in range(1, N_DEV):
            p = (my_pos + o) % N_DEV
            rdma = pltpu.make_async_remote_copy(
                src_ref=stage_ref.at[p],
                dst_ref=rs_comm_ref.at[o - 1],
                send_sem=rs_send_sems.at[o - 1],
                recv_sem=rs_recv_sems.at[o - 1],
                device_id=(p,),
                device_id_type=pl.DeviceIdType.MESH,
            )
            rdma.start()
            rs_sends.append(rdma)

        final = stage_ref[my_pos].astype(jnp.float32)
        for o in range(1, N_DEV):
            recv = pltpu.make_async_remote_copy(
                src_ref=stage_ref.at[0],
                dst_ref=rs_comm_ref.at[o - 1],
                send_sem=rs_send_sems.at[o - 1],
                recv_sem=rs_recv_sems.at[o - 1],
                device_id=((my_pos + o) % N_DEV,),
                device_id_type=pl.DeviceIdType.MESH,
            )
            recv.wait_recv()
            final = final + rs_comm_ref[o - 1, :, :].astype(jnp.float32)
        ag_stage_ref[:, :] = final.astype(jnp.bfloat16)

        ag_sends = []
        for o in range(1, N_DEV):
            rdma = pltpu.make_async_remote_copy(
                src_ref=ag_stage_ref,
                dst_ref=ag_comm_ref.at[o - 1],
                send_sem=ag_send_sems.at[o - 1],
                recv_sem=ag_recv_sems.at[o - 1],
                device_id=((my_pos + o) % N_DEV,),
                device_id_type=pl.DeviceIdType.MESH,
            )
            rdma.start()
            ag_sends.append(rdma)

        out_ref[:, pl.ds(my_pos * hc, hc)] = final

        for o in range(1, N_DEV):
            recv = pltpu.make_async_remote_copy(
                src_ref=ag_stage_ref,
                dst_ref=ag_comm_ref.at[o - 1],
                send_sem=ag_send_sems.at[o - 1],
                recv_sem=ag_recv_sems.at[o - 1],
                device_id=((my_pos + o) % N_DEV,),
                device_id_type=pl.DeviceIdType.MESH,
            )
            recv.wait_recv()
            p = (my_pos - o) % N_DEV
            out_ref[:, pl.ds(p * hc, hc)] = (
                ag_comm_ref[o - 1, :, :].astype(jnp.float32)
            )

        for rdma in rs_sends + ag_sends:
            rdma.wait_send()

    return pl.pallas_call(
        body,
        out_shape=jax.ShapeDtypeStruct((n, h), jnp.float32),
        in_specs=[
            pl.BlockSpec(memory_space=pltpu.VMEM),
            pl.BlockSpec(memory_space=pltpu.VMEM),
            pl.BlockSpec(memory_space=pltpu.VMEM),
        ],
        out_specs=pl.BlockSpec(memory_space=pltpu.VMEM),
        scratch_shapes=[
            pltpu.VMEM((N_DEV, n, hc), jnp.bfloat16),
            pltpu.VMEM((N_DEV - 1, n, hc), jnp.bfloat16),
            pltpu.VMEM((n, hc), jnp.bfloat16),
            pltpu.VMEM((N_DEV - 1, n, hc), jnp.bfloat16),
            pltpu.SemaphoreType.DMA((N_DEV - 1,)),
            pltpu.SemaphoreType.DMA((N_DEV - 1,)),
            pltpu.SemaphoreType.DMA((N_DEV - 1,)),
            pltpu.SemaphoreType.DMA((N_DEV - 1,)),
        ],
        compiler_params=pltpu.CompilerParams(collective_id=0),
    )(x, route_idx, expert_W)
